# baseline (speedup 1.0000x reference)
import functools

import jax
import jax.numpy as jnp
from jax import lax
from jax.experimental import pallas as pl
from jax.experimental.pallas import tpu as pltpu
from jax.experimental.pallas import tpu_sc as plsc

N = 10000
E = 320000
D = 128
NP = 10240
NC = 2
NS = 16
NW = NC * NS
GROWS = 2 * NP // NW
GCHUNKS = GROWS // 128
EPT = E // NS
CH = 48
G = 8
NG = 54
ECHUNKS = NG * G
EPTP = ECHUNKS * CH
NZ = 5


def _sc_gather_body(tbl_hbm, idx_hbm, out_hbm, idx_v, rows_v, sem):
    cid = lax.axis_index("c")
    sid = lax.axis_index("s")
    wid = cid * NS + sid
    base = wid * GROWS
    pltpu.sync_copy(idx_hbm.at[wid], idx_v)
    cps = [
        pltpu.async_copy(tbl_hbm.at[idx_v.at[k]],
                         rows_v.at[pl.ds(k * 128, 128)], sem)
        for k in range(GCHUNKS)
    ]
    for cp in cps:
        cp.wait()
    pltpu.sync_copy(rows_v, out_hbm.at[pl.ds(base, GROWS)])


def _sc_gather(embed, idx):
    mesh = plsc.VectorSubcoreMesh(core_axis_name="c", subcore_axis_name="s")
    f = functools.partial(
        pl.kernel,
        out_type=jax.ShapeDtypeStruct((2 * NP, D), jnp.float32),
        mesh=mesh,
        scratch_types=[
            pltpu.VMEM((GCHUNKS, 128), jnp.int32),
            pltpu.VMEM((GROWS, D), jnp.float32),
            pltpu.SemaphoreType.DMA,
        ],
    )(_sc_gather_body)
    return f(embed, idx)


def _tc_linear_body(xe_ref, w12_ref, ee_ref, w3_ref, bm_ref, y1_ref, y2_ref, t3_ref):
    x = xe_ref[...]
    y1_ref[...] = jnp.dot(x, w12_ref[:, :D], preferred_element_type=jnp.float32)
    y2_ref[...] = jnp.dot(x, w12_ref[:, D:], preferred_element_type=jnp.float32)

    @pl.when(pl.program_id(0) == 0)
    def _():
        t3_ref[...] = (
            jnp.dot(ee_ref[...], w3_ref[...], preferred_element_type=jnp.float32)
            + bm_ref[...]
        )


def _tc_linear(xe, w12, ee_pad, w3, bm2):
    bs = 2048
    grid = (2 * NP // bs,)
    return pl.pallas_call(
        _tc_linear_body,
        grid=grid,
        in_specs=[
            pl.BlockSpec((bs, D), lambda i: (i, 0)),
            pl.BlockSpec((D, 2 * D), lambda i: (0, 0)),
            pl.BlockSpec((32, D), lambda i: (0, 0)),
            pl.BlockSpec((D, D), lambda i: (0, 0)),
            pl.BlockSpec((1, D), lambda i: (0, 0)),
        ],
        out_specs=[
            pl.BlockSpec((bs, D), lambda i: (i, 0)),
            pl.BlockSpec((bs, D), lambda i: (i, 0)),
            pl.BlockSpec((32, D), lambda i: (0, 0)),
        ],
        out_shape=[
            jax.ShapeDtypeStruct((2 * NP, D), jnp.float32),
            jax.ShapeDtypeStruct((2 * NP, D), jnp.float32),
            jax.ShapeDtypeStruct((32, D), jnp.float32),
        ],
    )(xe, w12, ee_pad, w3, bm2)


def _compute(buf_a, buf_b, buf_c):
    def row(r, _):
        for cc in range(D // 16):
            sl = pl.ds(cc * 16, 16)
            buf_a[r, sl] = jnp.maximum(
                buf_a[r, sl] + buf_b[r, sl] + buf_c[r, sl], 0.0
            )
        return _

    lax.fori_loop(0, CH, row, None)


def _sc_mp_body(y1_hbm, y2_hbm, t3_hbm, dstg_hbm, srcg_hbm, attr_hbm, dsts_hbm,
                msg_hbm, acc, buf_a0, buf_a1, buf_b0, buf_b1, buf_c0, buf_c1,
                iv_d, iv_s, iv_a, iv_sc, sem0, sem1):
    cid = lax.axis_index("c")
    sid = lax.axis_index("s")
    bufs_a = (buf_a0, buf_a1)
    bufs_b = (buf_b0, buf_b1)
    bufs_c = (buf_c0, buf_c1)
    sems = (sem0, sem1)

    def zrow(r, _):
        for c in range(D // 16):
            buf_a0[r, pl.ds(c * 16, 16)] = jnp.zeros((16,), jnp.float32)
        return _

    lax.fori_loop(0, CH, zrow, None)
    for k in range(13):
        pltpu.sync_copy(buf_a0, acc.at[pl.ds(sid * GROWS + k * CH, CH)])
    pltpu.sync_copy(buf_a0.at[pl.ds(0, 16)],
                    acc.at[pl.ds(sid * GROWS + 13 * CH, 16)])
    plsc.subcore_barrier()

    def issue(gp, j, p):
        pltpu.async_copy(y1_hbm.at[iv_d.at[gp, j]], bufs_a[p], sems[p])
        pltpu.async_copy(y2_hbm.at[iv_s.at[gp, j]], bufs_b[p], sems[p])
        pltpu.async_copy(t3_hbm.at[iv_a.at[gp, j]], bufs_c[p], sems[p])

    def drain(gp, j, p):
        pltpu.make_async_copy(y1_hbm.at[iv_d.at[gp, j]], bufs_a[p], sems[p]).wait()
        pltpu.make_async_copy(y2_hbm.at[iv_s.at[gp, j]], bufs_b[p], sems[p]).wait()
        pltpu.make_async_copy(t3_hbm.at[iv_a.at[gp, j]], bufs_c[p], sems[p]).wait()

    def load_group(g, gp):
        pltpu.sync_copy(dstg_hbm.at[cid, sid, g], iv_d.at[gp])
        pltpu.sync_copy(srcg_hbm.at[cid, sid, g], iv_s.at[gp])
        pltpu.sync_copy(attr_hbm.at[cid, sid, g], iv_a.at[gp])
        pltpu.sync_copy(dsts_hbm.at[cid, sid, g], iv_sc.at[gp])

    def do_chunk(gp, j):
        p = j & 1
        pn = 1 - p
        if j < G - 1:
            issue(gp, j + 1, pn)
        drain(gp, j, p)
        _compute(bufs_a[p], bufs_b[p], bufs_c[p])
        pltpu.sync_copy(bufs_a[p], acc.at[iv_sc.at[gp, j]], add=True)

    load_group(0, 0)
    issue(0, 0, 0)

    def group_pair(g2, _):
        for gp in range(2):
            g = 2 * g2 + gp
            gpn = 1 - gp

            @pl.when(g + 1 < NG)
            def _():
                load_group(g + 1, gpn)

            for j in range(G - 1):
                do_chunk(gp, j)
            p = (G - 1) & 1
            pn = 1 - p

            @pl.when(g + 1 < NG)
            def _():
                issue(gpn, 0, pn)

            drain(gp, G - 1, p)
            _compute(bufs_a[p], bufs_b[p], bufs_c[p])
            pltpu.sync_copy(bufs_a[p], acc.at[iv_sc.at[gp, G - 1]], add=True)
        return _

    lax.fori_loop(0, NG // 2, group_pair, None)
    plsc.subcore_barrier()
    pltpu.sync_copy(
        acc.at[pl.ds(sid * GROWS, GROWS)],
        msg_hbm.at[pl.ds(cid * NP + sid * GROWS, GROWS)],
    )


def _sc_msgpass(y1, y2, t3, dstg, srcg, attr, dsts):
    mesh = plsc.VectorSubcoreMesh(core_axis_name="c", subcore_axis_name="s")
    f = functools.partial(
        pl.kernel,
        out_type=jax.ShapeDtypeStruct((2 * NP, D), jnp.float32),
        mesh=mesh,
        scratch_types=[
            pltpu.VMEM_SHARED((NP, D), jnp.float32),
            pltpu.VMEM((CH, D), jnp.float32),
            pltpu.VMEM((CH, D), jnp.float32),
            pltpu.VMEM((CH, D), jnp.float32),
            pltpu.VMEM((CH, D), jnp.float32),
            pltpu.VMEM((CH, D), jnp.float32),
            pltpu.VMEM((CH, D), jnp.float32),
            pltpu.VMEM((2, G, CH), jnp.int32),
            pltpu.VMEM((2, G, CH), jnp.int32),
            pltpu.VMEM((2, G, CH), jnp.int32),
            pltpu.VMEM((2, G, CH), jnp.int32),
            pltpu.SemaphoreType.DMA,
            pltpu.SemaphoreType.DMA,
        ],
    )(_sc_mp_body)
    return f(y1, y2, t3, dstg, srcg, attr, dsts)


def _flash_body(q_ref, kv_ref, o_ref, m_scr, l_scr, acc_scr):
    kb = pl.program_id(2)
    nkb = pl.num_programs(2)
    q = q_ref[0]
    kv = kv_ref[0]
    s = jnp.dot(q, kv.T, preferred_element_type=jnp.float32)
    col = kb * kv.shape[0] + lax.broadcasted_iota(jnp.int32, s.shape, 1)
    s = jnp.where(col < N, s, -1e30)

    @pl.when(kb == 0)
    def _():
        m = jnp.max(s, axis=1, keepdims=True)
        p = jnp.exp(s - m)
        m_scr[...] = m
        l_scr[...] = jnp.sum(p, axis=1, keepdims=True)
        acc_scr[...] = jnp.dot(p, kv, preferred_element_type=jnp.float32)

    @pl.when(kb != 0)
    def _():
        m_old = m_scr[...]
        m_new = jnp.maximum(m_old, jnp.max(s, axis=1, keepdims=True))
        alpha = jnp.exp(m_old - m_new)
        p = jnp.exp(s - m_new)
        m_scr[...] = m_new
        l_scr[...] = l_scr[...] * alpha + jnp.sum(p, axis=1, keepdims=True)
        acc_scr[...] = acc_scr[...] * alpha + jnp.dot(
            p, kv, preferred_element_type=jnp.float32
        )

    @pl.when(kb == nkb - 1)
    def _():
        o_ref[0] = acc_scr[...] / l_scr[...]


def _tc_flash(xe3):
    bq, bk = 256, 1024
    grid = (2, NP // bq, NP // bk)
    return pl.pallas_call(
        _flash_body,
        grid=grid,
        in_specs=[
            pl.BlockSpec((1, bq, D), lambda b, q, k: (b, q, 0)),
            pl.BlockSpec((1, bk, D), lambda b, q, k: (1 - b, k, 0)),
        ],
        out_specs=pl.BlockSpec((1, bq, D), lambda b, q, k: (b, q, 0)),
        out_shape=jax.ShapeDtypeStruct((2, NP, D), jnp.float32),
        scratch_shapes=[
            pltpu.VMEM((bq, 1), jnp.float32),
            pltpu.VMEM((bq, 1), jnp.float32),
            pltpu.VMEM((bq, D), jnp.float32),
        ],
        compiler_params=pltpu.CompilerParams(
            dimension_semantics=("parallel", "parallel", "arbitrary"),
        ),
    )(xe3, xe3)


def _gru_pool_body(h_ref, m_ref, a_ref, wim_ref, wiu_ref, whh_ref, bih_ref,
                   bhh_ref, wg_ref, bg_ref, o_ref, num_scr, den_scr):
    qb = pl.program_id(1)
    nqb = pl.num_programs(1)
    h = h_ref[0]
    m = m_ref[0]
    u = h - a_ref[0]
    gi = (
        jnp.dot(m, wim_ref[...], preferred_element_type=jnp.float32)
        + jnp.dot(u, wiu_ref[...], preferred_element_type=jnp.float32)
        + bih_ref[...]
    )
    gh = jnp.dot(h, whh_ref[...], preferred_element_type=jnp.float32) + bhh_ref[...]
    r = jax.nn.sigmoid(gi[:, :D] + gh[:, :D])
    z = jax.nn.sigmoid(gi[:, D:2 * D] + gh[:, D:2 * D])
    n = jnp.tanh(gi[:, 2 * D:] + r * gh[:, 2 * D:])
    hn = (1.0 - z) * n + z * h

    gate = jax.nn.sigmoid(
        jnp.sum(hn * wg_ref[...], axis=1, keepdims=True) + bg_ref[...]
    )
    rows = qb * hn.shape[0] + lax.broadcasted_iota(jnp.int32, gate.shape, 0)
    e = jnp.where(rows < N, jnp.exp(gate), 0.0)

    @pl.when(qb == 0)
    def _():
        num_scr[...] = jnp.zeros_like(num_scr)
        den_scr[...] = jnp.zeros_like(den_scr)

    num_scr[...] += lax.dot_general(
        e, hn, (((0,), (0,)), ((), ())), preferred_element_type=jnp.float32
    )
    den_scr[...] += jnp.sum(e, axis=0, keepdims=True)

    @pl.when(qb == nqb - 1)
    def _():
        o_ref[0] = num_scr[...] / den_scr[...]


def _tc_gru_pool(xe3, msg3, att3, wim, wiu, whh_t, bih2, bhh2, wg2, bg2):
    bs = 1024
    grid = (2, NP // bs)
    return pl.pallas_call(
        _gru_pool_body,
        grid=grid,
        in_specs=[
            pl.BlockSpec((1, bs, D), lambda b, q: (b, q, 0)),
            pl.BlockSpec((1, bs, D), lambda b, q: (b, q, 0)),
            pl.BlockSpec((1, bs, D), lambda b, q: (b, q, 0)),
            pl.BlockSpec((D, 3 * D), lambda b, q: (0, 0)),
            pl.BlockSpec((D, 3 * D), lambda b, q: (0, 0)),
            pl.BlockSpec((D, 3 * D), lambda b, q: (0, 0)),
            pl.BlockSpec((1, 3 * D), lambda b, q: (0, 0)),
            pl.BlockSpec((1, 3 * D), lambda b, q: (0, 0)),
            pl.BlockSpec((1, D), lambda b, q: (0, 0)),
            pl.BlockSpec((1, 1), lambda b, q: (0, 0)),
        ],
        out_specs=pl.BlockSpec((1, 1, D), lambda b, q: (b, 0, 0)),
        out_shape=jax.ShapeDtypeStruct((2, 1, D), jnp.float32),
        scratch_shapes=[
            pltpu.VMEM((1, D), jnp.float32),
            pltpu.VMEM((1, 1), jnp.float32),
        ],
        compiler_params=pltpu.CompilerParams(
            dimension_semantics=("parallel", "arbitrary"),
        ),
    )(xe3, msg3, att3, wim, wiu, whh_t, bih2, bhh2, wg2, bg2)


def _prep_edge_arrays(ei, ea, g):
    dst = ei[1].astype(jnp.int32).reshape(NS, EPT)
    src = ei[0].astype(jnp.int32).reshape(NS, EPT)
    attr = ea[:, 0].astype(jnp.int32).reshape(NS, EPT)
    pad = ((0, 0), (0, EPTP - EPT))
    dst_s = jnp.pad(dst, pad, constant_values=N)
    dst_g = jnp.pad(dst, pad, constant_values=N) + g * NP
    src_g = jnp.pad(src, pad, constant_values=N) + g * NP
    attr_p = jnp.pad(attr, pad, constant_values=0)
    shape = (NS, NG, G, CH)
    return (dst_g.reshape(shape), src_g.reshape(shape),
            attr_p.reshape(shape), dst_s.reshape(shape))


def kernel(x1, x2, edge_index1, edge_index2, edge_attr1, edge_attr2,
           embed, edge_embed, Wm, bm, W_ih, W_hh, b_ih, b_hh, Wg, bg):
    pad_n = NP - N
    idx = jnp.concatenate([
        jnp.pad(x1[:, 0].astype(jnp.int32), (0, pad_n)),
        jnp.pad(x2[:, 0].astype(jnp.int32), (0, pad_n)),
    ]).reshape(NW, GCHUNKS, 128)

    w12 = jnp.concatenate([Wm[:, :D].T, Wm[:, D:2 * D].T], axis=1)
    w3 = Wm[:, 2 * D:].T
    ee_pad = jnp.pad(edge_embed, ((0, 12), (0, 0)))
    bm2 = bm.reshape(1, D)

    e1 = _prep_edge_arrays(edge_index1, edge_attr1, 0)
    e2 = _prep_edge_arrays(edge_index2, edge_attr2, 1)
    dstg, srcg, attrp, dsts = (jnp.stack([a, b]) for a, b in zip(e1, e2))

    xe = _sc_gather(embed, idx)

    y1, y2, t3 = _tc_linear(xe, w12, ee_pad, w3, bm2)

    msg = _sc_msgpass(y1, y2, t3, dstg, srcg, attrp, dsts)

    xe3 = xe.reshape(2, NP, D)
    att3 = _tc_flash(xe3)

    wim = W_ih[:, :D].T
    wiu = W_ih[:, D:].T
    whh_t = W_hh.T
    hg = _tc_gru_pool(
        xe3, msg.reshape(2, NP, D), att3,
        wim, wiu, whh_t,
        b_ih.reshape(1, 3 * D), b_hh.reshape(1, 3 * D),
        Wg.reshape(1, D), bg.reshape(1, 1),
    )
    return (hg[0], hg[1])


# speedup vs baseline: 1.4076x; 1.4076x over previous
import functools

import jax
import jax.numpy as jnp
from jax import lax
from jax.experimental import pallas as pl
from jax.experimental.pallas import tpu as pltpu
from jax.experimental.pallas import tpu_sc as plsc

N = 10000
E = 320000
D = 128
NP = 10240
NC = 2
NS = 16
NW = NC * NS
GROWS = 2 * NP // NW
GCHUNKS = GROWS // 128
EPT = E // NS
CH = 40
G = 10
NG = 50
ECHUNKS = NG * G
NZ = 5


def _sc_gather_body(tbl_hbm, idx_hbm, out_hbm, idx_v, rows_v, sem):
    cid = lax.axis_index("c")
    sid = lax.axis_index("s")
    wid = cid * NS + sid
    base = wid * GROWS
    pltpu.sync_copy(idx_hbm.at[wid], idx_v)
    cps = [
        pltpu.async_copy(tbl_hbm.at[idx_v.at[k]],
                         rows_v.at[pl.ds(k * 128, 128)], sem)
        for k in range(GCHUNKS)
    ]
    for cp in cps:
        cp.wait()
    pltpu.sync_copy(rows_v, out_hbm.at[pl.ds(base, GROWS)])


def _sc_gather(embed, idx):
    mesh = plsc.VectorSubcoreMesh(core_axis_name="c", subcore_axis_name="s")
    f = functools.partial(
        pl.kernel,
        out_type=jax.ShapeDtypeStruct((2 * NP, D), jnp.float32),
        mesh=mesh,
        scratch_types=[
            pltpu.VMEM((GCHUNKS, 128), jnp.int32),
            pltpu.VMEM((GROWS, D), jnp.float32),
            pltpu.SemaphoreType.DMA,
        ],
    )(_sc_gather_body)
    return f(embed, idx)


def _tc_linear_body(xe_ref, w12_ref, ee_ref, w3_ref, bm_ref, y1_ref, y2_ref, t3_ref):
    x = xe_ref[...]
    y1_ref[...] = jnp.dot(x, w12_ref[:, :D], preferred_element_type=jnp.float32)
    y2_ref[...] = jnp.dot(x, w12_ref[:, D:], preferred_element_type=jnp.float32)

    @pl.when(pl.program_id(0) == 0)
    def _():
        t3_ref[...] = (
            jnp.dot(ee_ref[...], w3_ref[...], preferred_element_type=jnp.float32)
            + bm_ref[...]
        )


def _tc_linear(xe, w12, ee_pad, w3, bm2):
    bs = 2048
    grid = (2 * NP // bs,)
    return pl.pallas_call(
        _tc_linear_body,
        grid=grid,
        in_specs=[
            pl.BlockSpec((bs, D), lambda i: (i, 0)),
            pl.BlockSpec((D, 2 * D), lambda i: (0, 0)),
            pl.BlockSpec((32, D), lambda i: (0, 0)),
            pl.BlockSpec((D, D), lambda i: (0, 0)),
            pl.BlockSpec((1, D), lambda i: (0, 0)),
        ],
        out_specs=[
            pl.BlockSpec((bs, D), lambda i: (i, 0)),
            pl.BlockSpec((bs, D), lambda i: (i, 0)),
            pl.BlockSpec((32, D), lambda i: (0, 0)),
        ],
        out_shape=[
            jax.ShapeDtypeStruct((2 * NP, D), jnp.float32),
            jax.ShapeDtypeStruct((2 * NP, D), jnp.float32),
            jax.ShapeDtypeStruct((32, D), jnp.float32),
        ],
    )(xe, w12, ee_pad, w3, bm2)


def _sc_mp_body(z_hbm, ivg_hbm, ivsc_hbm, msg_hbm, acc, buf0, buf1,
                ivg, ivsc, sg0, sg1, ss0, ss1, si0, si1):
    cid = lax.axis_index("c")
    sid = lax.axis_index("s")
    bufs = (buf0, buf1)
    sgs = (sg0, sg1)
    sss = (ss0, ss1)
    sis = (si0, si1)

    def zrow(r, _):
        for c in range(D // 16):
            buf0[r, pl.ds(c * 16, 16)] = jnp.zeros((16,), jnp.float32)
        return _

    lax.fori_loop(0, 128, zrow, None)
    for k in range(NZ):
        pltpu.sync_copy(buf0, acc.at[pl.ds(sid * GROWS + k * 128, 128)])
    plsc.subcore_barrier()

    def issue_gather(gp, j, p):
        pltpu.async_copy(z_hbm.at[ivg.at[gp, j]],
                         bufs[p].at[pl.ds(0, 3 * CH)], sgs[p])

    def drain_gather(gp, j, p):
        pltpu.make_async_copy(z_hbm.at[ivg.at[gp, j]],
                              bufs[p].at[pl.ds(0, 3 * CH)], sgs[p]).wait()

    def issue_scatter(gp, j, p):
        pltpu.async_copy(bufs[p].at[pl.ds(0, CH)],
                         acc.at[ivsc.at[gp, j]], sss[p], add=True)

    def drain_scatter(gp, j, p):
        pltpu.make_async_copy(bufs[p].at[pl.ds(0, CH)],
                              acc.at[ivsc.at[gp, j]], sss[p]).wait()

    def prefetch_group(g, gp):
        pltpu.async_copy(ivg_hbm.at[cid, sid, g], ivg.at[gp], sis[gp])
        pltpu.async_copy(ivsc_hbm.at[cid, sid, g], ivsc.at[gp], sis[gp])

    def wait_group(g, gp):
        pltpu.make_async_copy(ivg_hbm.at[cid, sid, g], ivg.at[gp], sis[gp]).wait()
        pltpu.make_async_copy(ivsc_hbm.at[cid, sid, g], ivsc.at[gp], sis[gp]).wait()

    def compute(p):
        buf = bufs[p]

        def row(r, _):
            for cc in range(D // 16):
                sl = pl.ds(cc * 16, 16)
                buf[r, sl] = jnp.maximum(
                    buf[r, sl] + buf[CH + r, sl] + buf[2 * CH + r, sl], 0.0
                )
            return _

        lax.fori_loop(0, CH, row, None)

    pltpu.sync_copy(ivg_hbm.at[cid, sid, 0], ivg.at[0])
    pltpu.sync_copy(ivsc_hbm.at[cid, sid, 0], ivsc.at[0])
    issue_gather(0, 0, 0)

    def group_pair(g2, _):
        for gp in range(2):
            g = 2 * g2 + gp
            gpn = 1 - gp

            @pl.when(g + 1 < NG)
            def _():
                prefetch_group(g + 1, gpn)

            for j in range(G):
                p = j & 1
                pn = 1 - p
                drain_gather(gp, j, p)
                compute(p)
                if gp == 0 and j == 0:
                    @pl.when(g2 > 0)
                    def _():
                        drain_scatter(1, G - 1, pn)
                elif j == 0:
                    drain_scatter(0, G - 1, pn)
                else:
                    drain_scatter(gp, j - 1, pn)
                if j < G - 1:
                    issue_gather(gp, j + 1, pn)
                else:
                    if gp == 0:
                        wait_group(g + 1, gpn)
                        issue_gather(gpn, 0, pn)
                    else:
                        @pl.when(g2 + 1 < NG // 2)
                        def _():
                            wait_group(g + 1, gpn)
                            issue_gather(gpn, 0, pn)
                issue_scatter(gp, j, p)
        return _

    lax.fori_loop(0, NG // 2, group_pair, None)
    drain_scatter(1, G - 1, 1)
    plsc.subcore_barrier()
    pltpu.sync_copy(
        acc.at[pl.ds(sid * GROWS, GROWS)],
        msg_hbm.at[pl.ds(cid * NP + sid * GROWS, GROWS)],
    )


def _sc_msgpass(z, ivg_arr, ivsc_arr):
    mesh = plsc.VectorSubcoreMesh(core_axis_name="c", subcore_axis_name="s")
    f = functools.partial(
        pl.kernel,
        out_type=jax.ShapeDtypeStruct((2 * NP, D), jnp.float32),
        mesh=mesh,
        scratch_types=[
            pltpu.VMEM_SHARED((NP, D), jnp.float32),
            pltpu.VMEM((128, D), jnp.float32),
            pltpu.VMEM((128, D), jnp.float32),
            pltpu.VMEM((2, G, 3 * CH), jnp.int32),
            pltpu.VMEM((2, G, CH), jnp.int32),
            pltpu.SemaphoreType.DMA,
            pltpu.SemaphoreType.DMA,
            pltpu.SemaphoreType.DMA,
            pltpu.SemaphoreType.DMA,
            pltpu.SemaphoreType.DMA,
            pltpu.SemaphoreType.DMA,
        ],
    )(_sc_mp_body)
    return f(z, ivg_arr, ivsc_arr)


def _flash_body(q_ref, kv_ref, o_ref, m_scr, l_scr, acc_scr):
    kb = pl.program_id(2)
    nkb = pl.num_programs(2)
    q = q_ref[0]
    kv = kv_ref[0]
    s = jnp.dot(q, kv.T, preferred_element_type=jnp.float32)
    col = kb * kv.shape[0] + lax.broadcasted_iota(jnp.int32, s.shape, 1)
    s = jnp.where(col < N, s, -1e30)

    @pl.when(kb == 0)
    def _():
        m = jnp.max(s, axis=1, keepdims=True)
        p = jnp.exp(s - m)
        m_scr[...] = m
        l_scr[...] = jnp.sum(p, axis=1, keepdims=True)
        acc_scr[...] = jnp.dot(p, kv, preferred_element_type=jnp.float32)

    @pl.when(kb != 0)
    def _():
        m_old = m_scr[...]
        m_new = jnp.maximum(m_old, jnp.max(s, axis=1, keepdims=True))
        alpha = jnp.exp(m_old - m_new)
        p = jnp.exp(s - m_new)
        m_scr[...] = m_new
        l_scr[...] = l_scr[...] * alpha + jnp.sum(p, axis=1, keepdims=True)
        acc_scr[...] = acc_scr[...] * alpha + jnp.dot(
            p, kv, preferred_element_type=jnp.float32
        )

    @pl.when(kb == nkb - 1)
    def _():
        o_ref[0] = acc_scr[...] / l_scr[...]


def _tc_flash(xe3):
    bq, bk = 256, 1024
    grid = (2, NP // bq, NP // bk)
    return pl.pallas_call(
        _flash_body,
        grid=grid,
        in_specs=[
            pl.BlockSpec((1, bq, D), lambda b, q, k: (b, q, 0)),
            pl.BlockSpec((1, bk, D), lambda b, q, k: (1 - b, k, 0)),
        ],
        out_specs=pl.BlockSpec((1, bq, D), lambda b, q, k: (b, q, 0)),
        out_shape=jax.ShapeDtypeStruct((2, NP, D), jnp.float32),
        scratch_shapes=[
            pltpu.VMEM((bq, 1), jnp.float32),
            pltpu.VMEM((bq, 1), jnp.float32),
            pltpu.VMEM((bq, D), jnp.float32),
        ],
        compiler_params=pltpu.CompilerParams(
            dimension_semantics=("parallel", "parallel", "arbitrary"),
        ),
    )(xe3, xe3)


def _gru_pool_body(h_ref, m_ref, a_ref, wim_ref, wiu_ref, whh_ref, bih_ref,
                   bhh_ref, wg_ref, bg_ref, o_ref, num_scr, den_scr):
    qb = pl.program_id(1)
    nqb = pl.num_programs(1)
    h = h_ref[0]
    m = m_ref[0]
    u = h - a_ref[0]
    gi = (
        jnp.dot(m, wim_ref[...], preferred_element_type=jnp.float32)
        + jnp.dot(u, wiu_ref[...], preferred_element_type=jnp.float32)
        + bih_ref[...]
    )
    gh = jnp.dot(h, whh_ref[...], preferred_element_type=jnp.float32) + bhh_ref[...]
    r = jax.nn.sigmoid(gi[:, :D] + gh[:, :D])
    z = jax.nn.sigmoid(gi[:, D:2 * D] + gh[:, D:2 * D])
    n = jnp.tanh(gi[:, 2 * D:] + r * gh[:, 2 * D:])
    hn = (1.0 - z) * n + z * h

    gate = jax.nn.sigmoid(
        jnp.sum(hn * wg_ref[...], axis=1, keepdims=True) + bg_ref[...]
    )
    rows = qb * hn.shape[0] + lax.broadcasted_iota(jnp.int32, gate.shape, 0)
    e = jnp.where(rows < N, jnp.exp(gate), 0.0)

    @pl.when(qb == 0)
    def _():
        num_scr[...] = jnp.zeros_like(num_scr)
        den_scr[...] = jnp.zeros_like(den_scr)

    num_scr[...] += lax.dot_general(
        e, hn, (((0,), (0,)), ((), ())), preferred_element_type=jnp.float32
    )
    den_scr[...] += jnp.sum(e, axis=0, keepdims=True)

    @pl.when(qb == nqb - 1)
    def _():
        o_ref[0] = num_scr[...] / den_scr[...]


def _tc_gru_pool(xe3, msg3, att3, wim, wiu, whh_t, bih2, bhh2, wg2, bg2):
    bs = 1024
    grid = (2, NP // bs)
    return pl.pallas_call(
        _gru_pool_body,
        grid=grid,
        in_specs=[
            pl.BlockSpec((1, bs, D), lambda b, q: (b, q, 0)),
            pl.BlockSpec((1, bs, D), lambda b, q: (b, q, 0)),
            pl.BlockSpec((1, bs, D), lambda b, q: (b, q, 0)),
            pl.BlockSpec((D, 3 * D), lambda b, q: (0, 0)),
            pl.BlockSpec((D, 3 * D), lambda b, q: (0, 0)),
            pl.BlockSpec((D, 3 * D), lambda b, q: (0, 0)),
            pl.BlockSpec((1, 3 * D), lambda b, q: (0, 0)),
            pl.BlockSpec((1, 3 * D), lambda b, q: (0, 0)),
            pl.BlockSpec((1, D), lambda b, q: (0, 0)),
            pl.BlockSpec((1, 1), lambda b, q: (0, 0)),
        ],
        out_specs=pl.BlockSpec((1, 1, D), lambda b, q: (b, 0, 0)),
        out_shape=jax.ShapeDtypeStruct((2, 1, D), jnp.float32),
        scratch_shapes=[
            pltpu.VMEM((1, D), jnp.float32),
            pltpu.VMEM((1, 1), jnp.float32),
        ],
        compiler_params=pltpu.CompilerParams(
            dimension_semantics=("parallel", "arbitrary"),
        ),
    )(xe3, msg3, att3, wim, wiu, whh_t, bih2, bhh2, wg2, bg2)


def _prep_edge_arrays(ei, ea, g):
    dst = ei[1].astype(jnp.int32).reshape(NS, NG, G, CH)
    src = ei[0].astype(jnp.int32).reshape(NS, NG, G, CH)
    attr = ea[:, 0].astype(jnp.int32).reshape(NS, NG, G, CH)
    ivg = jnp.concatenate(
        [dst + g * NP, src + (2 * NP + g * NP), attr + 4 * NP], axis=-1
    )
    return ivg, dst


def kernel(x1, x2, edge_index1, edge_index2, edge_attr1, edge_attr2,
           embed, edge_embed, Wm, bm, W_ih, W_hh, b_ih, b_hh, Wg, bg):
    pad_n = NP - N
    idx = jnp.concatenate([
        jnp.pad(x1[:, 0].astype(jnp.int32), (0, pad_n)),
        jnp.pad(x2[:, 0].astype(jnp.int32), (0, pad_n)),
    ]).reshape(NW, GCHUNKS, 128)

    w12 = jnp.concatenate([Wm[:, :D].T, Wm[:, D:2 * D].T], axis=1)
    w3 = Wm[:, 2 * D:].T
    ee_pad = jnp.pad(edge_embed, ((0, 12), (0, 0)))
    bm2 = bm.reshape(1, D)

    ivg1, dsts1 = _prep_edge_arrays(edge_index1, edge_attr1, 0)
    ivg2, dsts2 = _prep_edge_arrays(edge_index2, edge_attr2, 1)
    ivg_arr = jnp.stack([ivg1, ivg2])
    ivsc_arr = jnp.stack([dsts1, dsts2])

    xe = _sc_gather(embed, idx)

    y1, y2, t3 = _tc_linear(xe, w12, ee_pad, w3, bm2)

    z = jnp.concatenate([y1, y2, t3])
    msg = _sc_msgpass(z, ivg_arr, ivsc_arr)

    xe3 = xe.reshape(2, NP, D)
    att3 = _tc_flash(xe3)

    wim = W_ih[:, :D].T
    wiu = W_ih[:, D:].T
    whh_t = W_hh.T
    hg = _tc_gru_pool(
        xe3, msg.reshape(2, NP, D), att3,
        wim, wiu, whh_t,
        b_ih.reshape(1, 3 * D), b_hh.reshape(1, 3 * D),
        Wg.reshape(1, D), bg.reshape(1, 1),
    )
    return (hg[0], hg[1])


# speedup vs baseline: 1.6125x; 1.1455x over previous
import functools

import jax
import jax.numpy as jnp
from jax import lax
from jax.experimental import pallas as pl
from jax.experimental.pallas import tpu as pltpu
from jax.experimental.pallas import tpu_sc as plsc

N = 10000
E = 320000
D = 128
NP = 10240
NC = 2
NS = 16
NW = NC * NS
GROWS = 2 * NP // NW
GCHUNKS = GROWS // 128
EPT = E // NS
CH = 48
G = 2
NG = 210
ECHUNKS = NG * G
EPTP = ECHUNKS * CH
NZ = 5


def _sc_gather_body(tbl_hbm, idx_hbm, out_hbm, idx_v, rows_v, sem):
    cid = lax.axis_index("c")
    sid = lax.axis_index("s")
    wid = cid * NS + sid
    base = wid * GROWS
    pltpu.sync_copy(idx_hbm.at[wid], idx_v)
    cps = [
        pltpu.async_copy(tbl_hbm.at[idx_v.at[k]],
                         rows_v.at[pl.ds(k * 128, 128)], sem)
        for k in range(GCHUNKS)
    ]
    for cp in cps:
        cp.wait()
    pltpu.sync_copy(rows_v, out_hbm.at[pl.ds(base, GROWS)])


def _sc_gather(embed, idx):
    mesh = plsc.VectorSubcoreMesh(core_axis_name="c", subcore_axis_name="s")
    f = functools.partial(
        pl.kernel,
        out_type=jax.ShapeDtypeStruct((2 * NP, D), jnp.float32),
        mesh=mesh,
        scratch_types=[
            pltpu.VMEM((GCHUNKS, 128), jnp.int32),
            pltpu.VMEM((GROWS, D), jnp.float32),
            pltpu.SemaphoreType.DMA,
        ],
    )(_sc_gather_body)
    return f(embed, idx)


def _tc_linear_body(xe_ref, w12_ref, ee_ref, w3_ref, bm_ref, y1_ref, y2_ref, t3_ref):
    x = xe_ref[...]
    y1_ref[...] = jnp.dot(x, w12_ref[:, :D], preferred_element_type=jnp.float32)
    y2_ref[...] = jnp.dot(x, w12_ref[:, D:], preferred_element_type=jnp.float32)

    @pl.when(pl.program_id(0) == 0)
    def _():
        t3_ref[...] = (
            jnp.dot(ee_ref[...], w3_ref[...], preferred_element_type=jnp.float32)
            + bm_ref[...]
        )


def _tc_linear(xe, w12, ee_pad, w3, bm2):
    bs = 2048
    grid = (2 * NP // bs,)
    return pl.pallas_call(
        _tc_linear_body,
        grid=grid,
        in_specs=[
            pl.BlockSpec((bs, D), lambda i: (i, 0)),
            pl.BlockSpec((D, 2 * D), lambda i: (0, 0)),
            pl.BlockSpec((32, D), lambda i: (0, 0)),
            pl.BlockSpec((D, D), lambda i: (0, 0)),
            pl.BlockSpec((1, D), lambda i: (0, 0)),
        ],
        out_specs=[
            pl.BlockSpec((bs, D), lambda i: (i, 0)),
            pl.BlockSpec((bs, D), lambda i: (i, 0)),
            pl.BlockSpec((32, D), lambda i: (0, 0)),
        ],
        out_shape=[
            jax.ShapeDtypeStruct((2 * NP, D), jnp.float32),
            jax.ShapeDtypeStruct((2 * NP, D), jnp.float32),
            jax.ShapeDtypeStruct((32, D), jnp.float32),
        ],
    )(xe, w12, ee_pad, w3, bm2)


def _sc_mp_body(z_hbm, t3_hbm, ivg_hbm, iva_hbm, ivsc_hbm, msg_hbm, acc,
                buf0, buf1, t3_v, ivg, iva, ivsc, sg0, sg1, ss0, ss1, si0, si1):
    cid = lax.axis_index("c")
    sid = lax.axis_index("s")
    bufs = (buf0, buf1)
    sgs = (sg0, sg1)
    sss = (ss0, ss1)
    sis = (si0, si1)

    def zrow(r, _):
        for c in range(D // 16):
            buf0[r, pl.ds(c * 16, 16)] = jnp.zeros((16,), jnp.float32)
        return _

    lax.fori_loop(0, 2 * CH, zrow, None)
    for k in range(6):
        pltpu.sync_copy(buf0, acc.at[pl.ds(sid * GROWS + k * 2 * CH, 2 * CH)])
    pltpu.sync_copy(buf0.at[pl.ds(0, GROWS - 6 * 2 * CH)],
                    acc.at[pl.ds(sid * GROWS + 6 * 2 * CH, GROWS - 6 * 2 * CH)])
    pltpu.sync_copy(t3_hbm.at[pl.ds(0, 24)], t3_v)
    plsc.subcore_barrier()

    def issue_gather(gp, j, p):
        pltpu.async_copy(z_hbm.at[ivg.at[gp, j]],
                         bufs[p].at[pl.ds(0, 2 * CH)], sgs[p])

    def drain_gather(gp, j, p):
        pltpu.make_async_copy(z_hbm.at[ivg.at[gp, j]],
                              bufs[p].at[pl.ds(0, 2 * CH)], sgs[p]).wait()

    def issue_scatter(gp, j, p):
        pltpu.async_copy(bufs[p].at[pl.ds(0, CH)],
                         acc.at[ivsc.at[gp, j]], sss[p], add=True)

    def drain_scatter(gp, j, p):
        pltpu.make_async_copy(bufs[p].at[pl.ds(0, CH)],
                              acc.at[ivsc.at[gp, j]], sss[p]).wait()

    def prefetch_group(g, gp):
        pltpu.async_copy(ivg_hbm.at[cid, sid, g], ivg.at[gp], sis[gp])
        pltpu.async_copy(iva_hbm.at[cid, sid, g], iva.at[gp], sis[gp])
        pltpu.async_copy(ivsc_hbm.at[cid, sid, g], ivsc.at[gp], sis[gp])

    def wait_group(g, gp):
        pltpu.make_async_copy(ivg_hbm.at[cid, sid, g], ivg.at[gp], sis[gp]).wait()
        pltpu.make_async_copy(iva_hbm.at[cid, sid, g], iva.at[gp], sis[gp]).wait()
        pltpu.make_async_copy(ivsc_hbm.at[cid, sid, g], ivsc.at[gp], sis[gp]).wait()

    def compute(gp, j, p):
        buf = bufs[p]

        def rgroup(rg, _):
            base = rg * 16
            av = iva[gp, j, pl.ds(base, 16)]
            for k in range(16):
                a = av[k]
                r = base + k
                for cc in range(D // 16):
                    sl = pl.ds(cc * 16, 16)
                    buf[r, sl] = jnp.maximum(
                        buf[r, sl] + buf[CH + r, sl] + t3_v[a, sl], 0.0
                    )
            return _

        lax.fori_loop(0, CH // 16, rgroup, None)

    pltpu.sync_copy(ivg_hbm.at[cid, sid, 0], ivg.at[0])
    pltpu.sync_copy(iva_hbm.at[cid, sid, 0], iva.at[0])
    pltpu.sync_copy(ivsc_hbm.at[cid, sid, 0], ivsc.at[0])
    issue_gather(0, 0, 0)

    def group_pair(g2, _):
        for gp in range(2):
            g = 2 * g2 + gp
            gpn = 1 - gp

            @pl.when(g + 1 < NG)
            def _():
                prefetch_group(g + 1, gpn)

            for j in range(G):
                p = j & 1
                pn = 1 - p
                drain_gather(gp, j, p)
                compute(gp, j, p)
                if gp == 0 and j == 0:
                    @pl.when(g2 > 0)
                    def _():
                        drain_scatter(1, G - 1, pn)
                elif j == 0:
                    drain_scatter(0, G - 1, pn)
                else:
                    drain_scatter(gp, j - 1, pn)
                if j < G - 1:
                    issue_gather(gp, j + 1, pn)
                else:
                    if gp == 0:
                        wait_group(g + 1, gpn)
                        issue_gather(gpn, 0, pn)
                    else:
                        @pl.when(g2 + 1 < NG // 2)
                        def _():
                            wait_group(g + 1, gpn)
                            issue_gather(gpn, 0, pn)
                issue_scatter(gp, j, p)
        return _

    lax.fori_loop(0, NG // 2, group_pair, None)
    drain_scatter(1, G - 1, (G - 1) & 1)
    plsc.subcore_barrier()
    pltpu.sync_copy(
        acc.at[pl.ds(sid * GROWS, GROWS)],
        msg_hbm.at[pl.ds(cid * NP + sid * GROWS, GROWS)],
    )


def _sc_msgpass(z, t3, ivg_arr, iva_arr, ivsc_arr):
    mesh = plsc.VectorSubcoreMesh(core_axis_name="c", subcore_axis_name="s")
    f = functools.partial(
        pl.kernel,
        out_type=jax.ShapeDtypeStruct((2 * NP, D), jnp.float32),
        mesh=mesh,
        scratch_types=[
            pltpu.VMEM_SHARED((NP, D), jnp.float32),
            pltpu.VMEM((2 * CH, D), jnp.float32),
            pltpu.VMEM((2 * CH, D), jnp.float32),
            pltpu.VMEM((24, D), jnp.float32),
            pltpu.VMEM((2, G, 2 * CH), jnp.int32),
            pltpu.VMEM((2, G, CH), jnp.int32),
            pltpu.VMEM((2, G, CH), jnp.int32),
            pltpu.SemaphoreType.DMA,
            pltpu.SemaphoreType.DMA,
            pltpu.SemaphoreType.DMA,
            pltpu.SemaphoreType.DMA,
            pltpu.SemaphoreType.DMA,
            pltpu.SemaphoreType.DMA,
        ],
    )(_sc_mp_body)
    return f(z, t3, ivg_arr, iva_arr, ivsc_arr)


def _flash_body(q_ref, kv_ref, o_ref, m_scr, l_scr, acc_scr):
    kb = pl.program_id(2)
    nkb = pl.num_programs(2)
    q = q_ref[0]
    kv = kv_ref[0]
    s = jnp.dot(q, kv.T, preferred_element_type=jnp.float32)
    col = kb * kv.shape[0] + lax.broadcasted_iota(jnp.int32, s.shape, 1)
    s = jnp.where(col < N, s, -1e30)

    @pl.when(kb == 0)
    def _():
        m = jnp.max(s, axis=1, keepdims=True)
        p = jnp.exp(s - m)
        m_scr[...] = m
        l_scr[...] = jnp.sum(p, axis=1, keepdims=True)
        acc_scr[...] = jnp.dot(p, kv, preferred_element_type=jnp.float32)

    @pl.when(kb != 0)
    def _():
        m_old = m_scr[...]
        m_new = jnp.maximum(m_old, jnp.max(s, axis=1, keepdims=True))
        alpha = jnp.exp(m_old - m_new)
        p = jnp.exp(s - m_new)
        m_scr[...] = m_new
        l_scr[...] = l_scr[...] * alpha + jnp.sum(p, axis=1, keepdims=True)
        acc_scr[...] = acc_scr[...] * alpha + jnp.dot(
            p, kv, preferred_element_type=jnp.float32
        )

    @pl.when(kb == nkb - 1)
    def _():
        o_ref[0] = acc_scr[...] / l_scr[...]


def _tc_flash(xe3):
    bq, bk = 256, 1024
    grid = (2, NP // bq, NP // bk)
    return pl.pallas_call(
        _flash_body,
        grid=grid,
        in_specs=[
            pl.BlockSpec((1, bq, D), lambda b, q, k: (b, q, 0)),
            pl.BlockSpec((1, bk, D), lambda b, q, k: (1 - b, k, 0)),
        ],
        out_specs=pl.BlockSpec((1, bq, D), lambda b, q, k: (b, q, 0)),
        out_shape=jax.ShapeDtypeStruct((2, NP, D), jnp.float32),
        scratch_shapes=[
            pltpu.VMEM((bq, 1), jnp.float32),
            pltpu.VMEM((bq, 1), jnp.float32),
            pltpu.VMEM((bq, D), jnp.float32),
        ],
        compiler_params=pltpu.CompilerParams(
            dimension_semantics=("parallel", "parallel", "arbitrary"),
        ),
    )(xe3, xe3)


def _gru_pool_body(h_ref, m_ref, a_ref, wim_ref, wiu_ref, whh_ref, bih_ref,
                   bhh_ref, wg_ref, bg_ref, o_ref, num_scr, den_scr):
    qb = pl.program_id(1)
    nqb = pl.num_programs(1)
    h = h_ref[0]
    m = m_ref[0]
    u = h - a_ref[0]
    gi = (
        jnp.dot(m, wim_ref[...], preferred_element_type=jnp.float32)
        + jnp.dot(u, wiu_ref[...], preferred_element_type=jnp.float32)
        + bih_ref[...]
    )
    gh = jnp.dot(h, whh_ref[...], preferred_element_type=jnp.float32) + bhh_ref[...]
    r = jax.nn.sigmoid(gi[:, :D] + gh[:, :D])
    z = jax.nn.sigmoid(gi[:, D:2 * D] + gh[:, D:2 * D])
    n = jnp.tanh(gi[:, 2 * D:] + r * gh[:, 2 * D:])
    hn = (1.0 - z) * n + z * h

    gate = jax.nn.sigmoid(
        jnp.sum(hn * wg_ref[...], axis=1, keepdims=True) + bg_ref[...]
    )
    rows = qb * hn.shape[0] + lax.broadcasted_iota(jnp.int32, gate.shape, 0)
    e = jnp.where(rows < N, jnp.exp(gate), 0.0)

    @pl.when(qb == 0)
    def _():
        num_scr[...] = jnp.zeros_like(num_scr)
        den_scr[...] = jnp.zeros_like(den_scr)

    num_scr[...] += lax.dot_general(
        e, hn, (((0,), (0,)), ((), ())), preferred_element_type=jnp.float32
    )
    den_scr[...] += jnp.sum(e, axis=0, keepdims=True)

    @pl.when(qb == nqb - 1)
    def _():
        o_ref[0] = num_scr[...] / den_scr[...]


def _tc_gru_pool(xe3, msg3, att3, wim, wiu, whh_t, bih2, bhh2, wg2, bg2):
    bs = 1024
    grid = (2, NP // bs)
    return pl.pallas_call(
        _gru_pool_body,
        grid=grid,
        in_specs=[
            pl.BlockSpec((1, bs, D), lambda b, q: (b, q, 0)),
            pl.BlockSpec((1, bs, D), lambda b, q: (b, q, 0)),
            pl.BlockSpec((1, bs, D), lambda b, q: (b, q, 0)),
            pl.BlockSpec((D, 3 * D), lambda b, q: (0, 0)),
            pl.BlockSpec((D, 3 * D), lambda b, q: (0, 0)),
            pl.BlockSpec((D, 3 * D), lambda b, q: (0, 0)),
            pl.BlockSpec((1, 3 * D), lambda b, q: (0, 0)),
            pl.BlockSpec((1, 3 * D), lambda b, q: (0, 0)),
            pl.BlockSpec((1, D), lambda b, q: (0, 0)),
            pl.BlockSpec((1, 1), lambda b, q: (0, 0)),
        ],
        out_specs=pl.BlockSpec((1, 1, D), lambda b, q: (b, 0, 0)),
        out_shape=jax.ShapeDtypeStruct((2, 1, D), jnp.float32),
        scratch_shapes=[
            pltpu.VMEM((1, D), jnp.float32),
            pltpu.VMEM((1, 1), jnp.float32),
        ],
        compiler_params=pltpu.CompilerParams(
            dimension_semantics=("parallel", "arbitrary"),
        ),
    )(xe3, msg3, att3, wim, wiu, whh_t, bih2, bhh2, wg2, bg2)


def _prep_edge_arrays(ei, ea, g):
    pad = ((0, 0), (0, EPTP - EPT))
    dst = jnp.pad(ei[1].astype(jnp.int32).reshape(NS, EPT), pad,
                  constant_values=N)
    src = jnp.pad(ei[0].astype(jnp.int32).reshape(NS, EPT), pad,
                  constant_values=N)
    attr = jnp.pad(ea[:, 0].astype(jnp.int32).reshape(NS, EPT), pad,
                   constant_values=0)
    shape = (NS, NG, G, CH)
    dst = dst.reshape(shape)
    src = src.reshape(shape)
    ivg = jnp.concatenate(
        [dst + g * NP, src + (2 * NP + g * NP)], axis=-1
    )
    return ivg, attr.reshape(shape), dst


def kernel(x1, x2, edge_index1, edge_index2, edge_attr1, edge_attr2,
           embed, edge_embed, Wm, bm, W_ih, W_hh, b_ih, b_hh, Wg, bg):
    pad_n = NP - N
    idx = jnp.concatenate([
        jnp.pad(x1[:, 0].astype(jnp.int32), (0, pad_n)),
        jnp.pad(x2[:, 0].astype(jnp.int32), (0, pad_n)),
    ]).reshape(NW, GCHUNKS, 128)

    w12 = jnp.concatenate([Wm[:, :D].T, Wm[:, D:2 * D].T], axis=1)
    w3 = Wm[:, 2 * D:].T
    ee_pad = jnp.pad(edge_embed, ((0, 12), (0, 0)))
    bm2 = bm.reshape(1, D)

    ivg1, iva1, dsts1 = _prep_edge_arrays(edge_index1, edge_attr1, 0)
    ivg2, iva2, dsts2 = _prep_edge_arrays(edge_index2, edge_attr2, 1)
    ivg_arr = jnp.stack([ivg1, ivg2])
    iva_arr = jnp.stack([iva1, iva2])
    ivsc_arr = jnp.stack([dsts1, dsts2])

    xe = _sc_gather(embed, idx)

    y1, y2, t3 = _tc_linear(xe, w12, ee_pad, w3, bm2)

    z = jnp.concatenate([y1, y2])
    msg = _sc_msgpass(z, t3, ivg_arr, iva_arr, ivsc_arr)

    xe3 = xe.reshape(2, NP, D)
    att3 = _tc_flash(xe3)

    wim = W_ih[:, :D].T
    wiu = W_ih[:, D:].T
    whh_t = W_hh.T
    hg = _tc_gru_pool(
        xe3, msg.reshape(2, NP, D), att3,
        wim, wiu, whh_t,
        b_ih.reshape(1, 3 * D), b_hh.reshape(1, 3 * D),
        Wg.reshape(1, D), bg.reshape(1, 1),
    )
    return (hg[0], hg[1])


# speedup vs baseline: 1.9981x; 1.2392x over previous
import functools

import jax
import jax.numpy as jnp
from jax import lax
from jax.experimental import pallas as pl
from jax.experimental.pallas import tpu as pltpu
from jax.experimental.pallas import tpu_sc as plsc

N = 10000
E = 320000
D = 128
NP = 10240
NC = 2
NS = 16
NW = NC * NS
GROWS = 2 * NP // NW
GCHUNKS = GROWS // 128
EPT = E // NS
CH = 48
G = 2
NG = 210
ECHUNKS = NG * G
EPTP = ECHUNKS * CH
NZ = 5


def _sc_gather_body(tbl_hbm, idx_hbm, out_hbm, idx_v, rows_v, sem):
    cid = lax.axis_index("c")
    sid = lax.axis_index("s")
    wid = cid * NS + sid
    base = wid * GROWS
    pltpu.sync_copy(idx_hbm.at[wid], idx_v)
    cps = [
        pltpu.async_copy(tbl_hbm.at[idx_v.at[k]],
                         rows_v.at[pl.ds(k * 128, 128)], sem)
        for k in range(GCHUNKS)
    ]
    for cp in cps:
        cp.wait()
    pltpu.sync_copy(rows_v, out_hbm.at[pl.ds(base, GROWS)])


def _sc_gather(embed, idx):
    mesh = plsc.VectorSubcoreMesh(core_axis_name="c", subcore_axis_name="s")
    f = functools.partial(
        pl.kernel,
        out_type=jax.ShapeDtypeStruct((2 * NP, D), jnp.float32),
        mesh=mesh,
        scratch_types=[
            pltpu.VMEM((GCHUNKS, 128), jnp.int32),
            pltpu.VMEM((GROWS, D), jnp.float32),
            pltpu.SemaphoreType.DMA,
        ],
    )(_sc_gather_body)
    return f(embed, idx)


def _tc_linear_body(xe_ref, w12_ref, ee_ref, w3_ref, bm_ref, y1_ref, y2_ref, t3_ref):
    x = xe_ref[...]
    y1_ref[...] = jnp.dot(x, w12_ref[:, :D], preferred_element_type=jnp.float32)
    y2_ref[...] = jnp.dot(x, w12_ref[:, D:], preferred_element_type=jnp.float32)

    @pl.when(pl.program_id(0) == 0)
    def _():
        t3_ref[...] = (
            jnp.dot(ee_ref[...], w3_ref[...], preferred_element_type=jnp.float32)
            + bm_ref[...]
        )


def _tc_linear(xe, w12, ee_pad, w3, bm2):
    bs = 2048
    grid = (2 * NP // bs,)
    return pl.pallas_call(
        _tc_linear_body,
        grid=grid,
        in_specs=[
            pl.BlockSpec((bs, D), lambda i: (i, 0)),
            pl.BlockSpec((D, 2 * D), lambda i: (0, 0)),
            pl.BlockSpec((32, D), lambda i: (0, 0)),
            pl.BlockSpec((D, D), lambda i: (0, 0)),
            pl.BlockSpec((1, D), lambda i: (0, 0)),
        ],
        out_specs=[
            pl.BlockSpec((bs, D), lambda i: (i, 0)),
            pl.BlockSpec((bs, D), lambda i: (i, 0)),
            pl.BlockSpec((32, D), lambda i: (0, 0)),
        ],
        out_shape=[
            jax.ShapeDtypeStruct((2 * NP, D), jnp.float32),
            jax.ShapeDtypeStruct((2 * NP, D), jnp.float32),
            jax.ShapeDtypeStruct((32, D), jnp.float32),
        ],
    )(xe, w12, ee_pad, w3, bm2)


def _sc_mp_body(z_hbm, t3_hbm, ivg_hbm, iva_hbm, ivsc_hbm, msg_hbm, acc,
                buf0, buf1, t3_v, ivg, iva, ivsc, sg0, sg1, ss0, ss1, si0, si1):
    cid = lax.axis_index("c")
    sid = lax.axis_index("s")
    bufs = (buf0, buf1)
    sgs = (sg0, sg1)
    sss = (ss0, ss1)
    sis = (si0, si1)

    def zrow(r, _):
        for c in range(D // 16):
            buf0[r, pl.ds(c * 16, 16)] = jnp.zeros((16,), jnp.float32)
        return _

    lax.fori_loop(0, 2 * CH, zrow, None)
    for k in range(6):
        pltpu.sync_copy(buf0, acc.at[pl.ds(sid * GROWS + k * 2 * CH, 2 * CH)])
    pltpu.sync_copy(buf0.at[pl.ds(0, GROWS - 6 * 2 * CH)],
                    acc.at[pl.ds(sid * GROWS + 6 * 2 * CH, GROWS - 6 * 2 * CH)])
    pltpu.sync_copy(t3_hbm.at[pl.ds(0, 24)], t3_v)
    plsc.subcore_barrier()

    def issue_gather(gp, j, p):
        pltpu.async_copy(z_hbm.at[ivg.at[gp, j]],
                         bufs[p].at[pl.ds(0, 2 * CH)], sgs[p])

    def drain_gather(gp, j, p):
        pltpu.make_async_copy(z_hbm.at[ivg.at[gp, j]],
                              bufs[p].at[pl.ds(0, 2 * CH)], sgs[p]).wait()

    def issue_scatter(gp, j, p):
        pltpu.async_copy(bufs[p].at[pl.ds(0, CH)],
                         acc.at[ivsc.at[gp, j]], sss[p], add=True)

    def drain_scatter(gp, j, p):
        pltpu.make_async_copy(bufs[p].at[pl.ds(0, CH)],
                              acc.at[ivsc.at[gp, j]], sss[p]).wait()

    def prefetch_group(g, gp):
        pltpu.async_copy(ivg_hbm.at[cid, sid, g], ivg.at[gp], sis[gp])
        pltpu.async_copy(iva_hbm.at[cid, sid, g], iva.at[gp], sis[gp])
        pltpu.async_copy(ivsc_hbm.at[cid, sid, g], ivsc.at[gp], sis[gp])

    def wait_group(g, gp):
        pltpu.make_async_copy(ivg_hbm.at[cid, sid, g], ivg.at[gp], sis[gp]).wait()
        pltpu.make_async_copy(iva_hbm.at[cid, sid, g], iva.at[gp], sis[gp]).wait()
        pltpu.make_async_copy(ivsc_hbm.at[cid, sid, g], ivsc.at[gp], sis[gp]).wait()

    def compute(gp, j, p):
        buf = bufs[p]

        def rgroup(rg, _):
            base = rg * 16
            av = iva[gp, j, pl.ds(base, 16)]
            for k in range(16):
                a = av[k]
                r = base + k
                for cc in range(D // 16):
                    sl = pl.ds(cc * 16, 16)
                    buf[r, sl] = jnp.maximum(
                        buf[r, sl] + buf[CH + r, sl] + t3_v[a, sl], 0.0
                    )
            return _

        lax.fori_loop(0, CH // 16, rgroup, None)

    pltpu.sync_copy(ivg_hbm.at[cid, sid, 0], ivg.at[0])
    pltpu.sync_copy(iva_hbm.at[cid, sid, 0], iva.at[0])
    pltpu.sync_copy(ivsc_hbm.at[cid, sid, 0], ivsc.at[0])
    issue_gather(0, 0, 0)

    def group_pair(g2, _):
        for gp in range(2):
            g = 2 * g2 + gp
            gpn = 1 - gp

            @pl.when(g + 1 < NG)
            def _():
                prefetch_group(g + 1, gpn)

            for j in range(G):
                p = j & 1
                pn = 1 - p
                drain_gather(gp, j, p)
                if gp == 0 and j == 0:
                    @pl.when(g2 > 0)
                    def _():
                        drain_scatter(1, G - 1, pn)
                elif j == 0:
                    drain_scatter(0, G - 1, pn)
                else:
                    drain_scatter(gp, j - 1, pn)
                if j < G - 1:
                    issue_gather(gp, j + 1, pn)
                else:
                    if gp == 0:
                        wait_group(g + 1, gpn)
                        issue_gather(gpn, 0, pn)
                    else:
                        @pl.when(g2 + 1 < NG // 2)
                        def _():
                            wait_group(g + 1, gpn)
                            issue_gather(gpn, 0, pn)
                compute(gp, j, p)
                issue_scatter(gp, j, p)
        return _

    lax.fori_loop(0, NG // 2, group_pair, None)
    drain_scatter(1, G - 1, (G - 1) & 1)
    plsc.subcore_barrier()
    pltpu.sync_copy(
        acc.at[pl.ds(sid * GROWS, GROWS)],
        msg_hbm.at[pl.ds(cid * NP + sid * GROWS, GROWS)],
    )


def _sc_msgpass(z, t3, ivg_arr, iva_arr, ivsc_arr):
    mesh = plsc.VectorSubcoreMesh(core_axis_name="c", subcore_axis_name="s")
    f = functools.partial(
        pl.kernel,
        out_type=jax.ShapeDtypeStruct((2 * NP, D), jnp.float32),
        mesh=mesh,
        scratch_types=[
            pltpu.VMEM_SHARED((NP, D), jnp.float32),
            pltpu.VMEM((2 * CH, D), jnp.float32),
            pltpu.VMEM((2 * CH, D), jnp.float32),
            pltpu.VMEM((24, D), jnp.float32),
            pltpu.VMEM((2, G, 2 * CH), jnp.int32),
            pltpu.VMEM((2, G, CH), jnp.int32),
            pltpu.VMEM((2, G, CH), jnp.int32),
            pltpu.SemaphoreType.DMA,
            pltpu.SemaphoreType.DMA,
            pltpu.SemaphoreType.DMA,
            pltpu.SemaphoreType.DMA,
            pltpu.SemaphoreType.DMA,
            pltpu.SemaphoreType.DMA,
        ],
    )(_sc_mp_body)
    return f(z, t3, ivg_arr, iva_arr, ivsc_arr)


def _flash_body(q_ref, kv_ref, o_ref, m_scr, l_scr, acc_scr):
    kb = pl.program_id(2)
    nkb = pl.num_programs(2)
    q = q_ref[0]
    kv = kv_ref[0]
    s = jnp.dot(q, kv.T, preferred_element_type=jnp.float32)
    col = kb * kv.shape[0] + lax.broadcasted_iota(jnp.int32, s.shape, 1)
    s = jnp.where(col < N, s, -1e30)

    @pl.when(kb == 0)
    def _():
        m = jnp.max(s, axis=1, keepdims=True)
        p = jnp.exp(s - m)
        m_scr[...] = m
        l_scr[...] = jnp.sum(p, axis=1, keepdims=True)
        acc_scr[...] = jnp.dot(p, kv, preferred_element_type=jnp.float32)

    @pl.when(kb != 0)
    def _():
        m_old = m_scr[...]
        m_new = jnp.maximum(m_old, jnp.max(s, axis=1, keepdims=True))
        alpha = jnp.exp(m_old - m_new)
        p = jnp.exp(s - m_new)
        m_scr[...] = m_new
        l_scr[...] = l_scr[...] * alpha + jnp.sum(p, axis=1, keepdims=True)
        acc_scr[...] = acc_scr[...] * alpha + jnp.dot(
            p, kv, preferred_element_type=jnp.float32
        )

    @pl.when(kb == nkb - 1)
    def _():
        o_ref[0] = acc_scr[...] / l_scr[...]


def _tc_flash(xe3):
    bq, bk = 256, 1024
    grid = (2, NP // bq, NP // bk)
    return pl.pallas_call(
        _flash_body,
        grid=grid,
        in_specs=[
            pl.BlockSpec((1, bq, D), lambda b, q, k: (b, q, 0)),
            pl.BlockSpec((1, bk, D), lambda b, q, k: (1 - b, k, 0)),
        ],
        out_specs=pl.BlockSpec((1, bq, D), lambda b, q, k: (b, q, 0)),
        out_shape=jax.ShapeDtypeStruct((2, NP, D), jnp.float32),
        scratch_shapes=[
            pltpu.VMEM((bq, 1), jnp.float32),
            pltpu.VMEM((bq, 1), jnp.float32),
            pltpu.VMEM((bq, D), jnp.float32),
        ],
        compiler_params=pltpu.CompilerParams(
            dimension_semantics=("parallel", "parallel", "arbitrary"),
        ),
    )(xe3, xe3)


def _gru_pool_body(h_ref, m_ref, a_ref, wim_ref, wiu_ref, whh_ref, bih_ref,
                   bhh_ref, wg_ref, bg_ref, o_ref, num_scr, den_scr):
    qb = pl.program_id(1)
    nqb = pl.num_programs(1)
    h = h_ref[0]
    m = m_ref[0]
    u = h - a_ref[0]
    gi = (
        jnp.dot(m, wim_ref[...], preferred_element_type=jnp.float32)
        + jnp.dot(u, wiu_ref[...], preferred_element_type=jnp.float32)
        + bih_ref[...]
    )
    gh = jnp.dot(h, whh_ref[...], preferred_element_type=jnp.float32) + bhh_ref[...]
    r = jax.nn.sigmoid(gi[:, :D] + gh[:, :D])
    z = jax.nn.sigmoid(gi[:, D:2 * D] + gh[:, D:2 * D])
    n = jnp.tanh(gi[:, 2 * D:] + r * gh[:, 2 * D:])
    hn = (1.0 - z) * n + z * h

    gate = jax.nn.sigmoid(
        jnp.sum(hn * wg_ref[...], axis=1, keepdims=True) + bg_ref[...]
    )
    rows = qb * hn.shape[0] + lax.broadcasted_iota(jnp.int32, gate.shape, 0)
    e = jnp.where(rows < N, jnp.exp(gate), 0.0)

    @pl.when(qb == 0)
    def _():
        num_scr[...] = jnp.zeros_like(num_scr)
        den_scr[...] = jnp.zeros_like(den_scr)

    num_scr[...] += lax.dot_general(
        e, hn, (((0,), (0,)), ((), ())), preferred_element_type=jnp.float32
    )
    den_scr[...] += jnp.sum(e, axis=0, keepdims=True)

    @pl.when(qb == nqb - 1)
    def _():
        o_ref[0] = num_scr[...] / den_scr[...]


def _tc_gru_pool(xe3, msg3, att3, wim, wiu, whh_t, bih2, bhh2, wg2, bg2):
    bs = 1024
    grid = (2, NP // bs)
    return pl.pallas_call(
        _gru_pool_body,
        grid=grid,
        in_specs=[
            pl.BlockSpec((1, bs, D), lambda b, q: (b, q, 0)),
            pl.BlockSpec((1, bs, D), lambda b, q: (b, q, 0)),
            pl.BlockSpec((1, bs, D), lambda b, q: (b, q, 0)),
            pl.BlockSpec((D, 3 * D), lambda b, q: (0, 0)),
            pl.BlockSpec((D, 3 * D), lambda b, q: (0, 0)),
            pl.BlockSpec((D, 3 * D), lambda b, q: (0, 0)),
            pl.BlockSpec((1, 3 * D), lambda b, q: (0, 0)),
            pl.BlockSpec((1, 3 * D), lambda b, q: (0, 0)),
            pl.BlockSpec((1, D), lambda b, q: (0, 0)),
            pl.BlockSpec((1, 1), lambda b, q: (0, 0)),
        ],
        out_specs=pl.BlockSpec((1, 1, D), lambda b, q: (b, 0, 0)),
        out_shape=jax.ShapeDtypeStruct((2, 1, D), jnp.float32),
        scratch_shapes=[
            pltpu.VMEM((1, D), jnp.float32),
            pltpu.VMEM((1, 1), jnp.float32),
        ],
        compiler_params=pltpu.CompilerParams(
            dimension_semantics=("parallel", "arbitrary"),
        ),
    )(xe3, msg3, att3, wim, wiu, whh_t, bih2, bhh2, wg2, bg2)


def _prep_edge_arrays(ei, ea, g):
    pad = ((0, 0), (0, EPTP - EPT))
    dst = jnp.pad(ei[1].astype(jnp.int32).reshape(NS, EPT), pad,
                  constant_values=N)
    src = jnp.pad(ei[0].astype(jnp.int32).reshape(NS, EPT), pad,
                  constant_values=N)
    attr = jnp.pad(ea[:, 0].astype(jnp.int32).reshape(NS, EPT), pad,
                   constant_values=0)
    shape = (NS, NG, G, CH)
    dst = dst.reshape(shape)
    src = src.reshape(shape)
    ivg = jnp.concatenate(
        [dst + g * NP, src + (2 * NP + g * NP)], axis=-1
    )
    return ivg, attr.reshape(shape), dst


def kernel(x1, x2, edge_index1, edge_index2, edge_attr1, edge_attr2,
           embed, edge_embed, Wm, bm, W_ih, W_hh, b_ih, b_hh, Wg, bg):
    pad_n = NP - N
    idx = jnp.concatenate([
        jnp.pad(x1[:, 0].astype(jnp.int32), (0, pad_n)),
        jnp.pad(x2[:, 0].astype(jnp.int32), (0, pad_n)),
    ]).reshape(NW, GCHUNKS, 128)

    w12 = jnp.concatenate([Wm[:, :D].T, Wm[:, D:2 * D].T], axis=1)
    w3 = Wm[:, 2 * D:].T
    ee_pad = jnp.pad(edge_embed, ((0, 12), (0, 0)))
    bm2 = bm.reshape(1, D)

    ivg1, iva1, dsts1 = _prep_edge_arrays(edge_index1, edge_attr1, 0)
    ivg2, iva2, dsts2 = _prep_edge_arrays(edge_index2, edge_attr2, 1)
    ivg_arr = jnp.stack([ivg1, ivg2])
    iva_arr = jnp.stack([iva1, iva2])
    ivsc_arr = jnp.stack([dsts1, dsts2])

    xe = _sc_gather(embed, idx)

    y1, y2, t3 = _tc_linear(xe, w12, ee_pad, w3, bm2)

    z = jnp.concatenate([y1, y2])
    msg = _sc_msgpass(z, t3, ivg_arr, iva_arr, ivsc_arr)

    xe3 = xe.reshape(2, NP, D)
    att3 = _tc_flash(xe3)

    wim = W_ih[:, :D].T
    wiu = W_ih[:, D:].T
    whh_t = W_hh.T
    hg = _tc_gru_pool(
        xe3, msg.reshape(2, NP, D), att3,
        wim, wiu, whh_t,
        b_ih.reshape(1, 3 * D), b_hh.reshape(1, 3 * D),
        Wg.reshape(1, D), bg.reshape(1, 1),
    )
    return (hg[0], hg[1])
